# baseline (speedup 1.0000x reference)
import jax
import jax.numpy as jnp
from jax import lax
from jax.experimental import pallas as pl
from jax.experimental.pallas import tpu as pltpu
from jax.experimental.pallas import tpu_sc as plsc

B, S, D = 2, 4096, 2048
K = 410
KPAD = 512
BS = 512
NSUB = 16
ROWS_PER_SUB = KPAD // NSUB


def _score_body(x_ref, w1t_ref, b1_ref, w2_ref, b2_ref, m_ref, o_ref):
    xb = x_ref[...].astype(jnp.bfloat16)
    h = lax.dot_general(xb, w1t_ref[...], (((1,), (1,)), ((), ())),
                        preferred_element_type=jnp.float32) + b1_ref[...]
    h = jnp.maximum(h, 0.0)
    hb = h.astype(jnp.bfloat16)
    s = lax.dot_general(w2_ref[...], hb, (((1,), (1,)), ((), ())),
                        preferred_element_type=jnp.float32)
    s = s + b2_ref[...]
    neg = jnp.finfo(jnp.float32).min
    o_ref[...] = jnp.where(m_ref[0] != 0, s, neg)[None]


def _scores_b(xb2d, w1t, b1, w2row, b2, mask3d):
    grid = S // BS
    return pl.pallas_call(
        _score_body,
        grid=(grid,),
        in_specs=[
            pl.BlockSpec((BS, D), lambda i: (i, 0)),
            pl.BlockSpec((D, D), lambda i: (0, 0)),
            pl.BlockSpec((1, D), lambda i: (0, 0)),
            pl.BlockSpec((1, D), lambda i: (0, 0)),
            pl.BlockSpec((1, 1), lambda i: (0, 0)),
            pl.BlockSpec((1, 1, BS), lambda i: (0, 0, i)),
        ],
        out_specs=pl.BlockSpec((1, 1, BS), lambda i: (0, 0, i)),
        out_shape=jax.ShapeDtypeStruct((1, 1, S), jnp.float32),
    )(xb2d, w1t, b1, w2row, b2, mask3d)


def _sortable_i32(bits):
    return jnp.where(
        bits >= 0, bits, jnp.bitwise_xor(jnp.bitwise_and(bits, 0x7FFFFFFF), -1)
    )


def _thresh_body(s_ref, thr_ref, quo_ref):
    nb = s_ref.shape[0]
    bits = lax.bitcast_convert_type(s_ref[...], jnp.int32)
    key = _sortable_i32(bits)
    imin = jnp.int32(-2147483648)

    def step(j, prefix_u):
        b = 31 - j
        cand_u = jnp.bitwise_or(prefix_u, jnp.left_shift(jnp.int32(1), b))
        cand_key = jnp.bitwise_xor(cand_u, imin)
        cnt = jnp.sum((key >= cand_key).astype(jnp.int32), axis=1, keepdims=True)
        return jnp.where(cnt >= K, cand_u, prefix_u)

    prefix_u = lax.fori_loop(0, 32, step, jnp.zeros((nb, 1), jnp.int32))
    t = jnp.bitwise_xor(prefix_u, imin)
    cnt_gt = jnp.sum((key > t).astype(jnp.int32), axis=1, keepdims=True)
    quota = K - cnt_gt
    t_f = lax.bitcast_convert_type(_sortable_i32(t), jnp.float32)
    thr_ref[...] = jnp.broadcast_to(t_f, (nb, 128))
    quo_ref[...] = jnp.broadcast_to(quota, (nb, 128))


def _threshold_b(scores2d):
    nb = scores2d.shape[0]
    return pl.pallas_call(
        _thresh_body,
        out_shape=[
            jax.ShapeDtypeStruct((nb, 128), jnp.float32),
            jax.ShapeDtypeStruct((nb, 128), jnp.int32),
        ],
    )(scores2d)


def _make_sc_body(bconst):
    def _sc_body(scores_hbm, thr_hbm, quo_hbm, hs_hbm,
                 idx_out, nsc_out, gath_out,
                 scores_v, thr_v, quo_v, idxraw_v, idxoff_v, nsc_v,
                 shared_idx, chunk_v, rows_v, sem):
        c = lax.axis_index("c")
        s = lax.axis_index("s")

        @pl.when(jnp.logical_and(c == 0, s == 0))
        def _compact():
            pltpu.sync_copy(scores_hbm.at[0], scores_v)
            pltpu.sync_copy(thr_hbm.at[0], thr_v)
            pltpu.sync_copy(quo_hbm.at[0], quo_v)

            def zero(j, carry):
                idxraw_v[pl.ds(j * 16, 16)] = jnp.zeros((16,), jnp.int32)
                nsc_v[pl.ds(j * 16, 16)] = jnp.zeros((16,), jnp.float32)
                return carry

            lax.fori_loop(0, KPAD // 16, zero, jnp.int32(0))

            def body(i, carry):
                off, eqc = carry
                tvec = thr_v[pl.ds(0, 16)]
                qvec = quo_v[pl.ds(0, 16)]
                iota = lax.iota(jnp.int32, 16)
                sv = scores_v[pl.ds(i * 16, 16)]
                selgt = sv > tvec
                seleq = sv == tvec
                eqi = jnp.where(seleq, jnp.full((16,), 1, jnp.int32),
                                jnp.zeros((16,), jnp.int32))
                rank = plsc.cumsum(eqi) - eqi + eqc
                sel = jnp.logical_or(selgt, jnp.logical_and(seleq, rank < qvec))
                plsc.store_compressed(idxraw_v.at[pl.ds(off, 16)],
                                      iota + i * 16, mask=sel)
                plsc.store_compressed(nsc_v.at[pl.ds(off, 16)], sv, mask=sel)
                seli = jnp.where(sel, jnp.full((16,), 1, jnp.int32),
                                 jnp.zeros((16,), jnp.int32))
                cnt = jnp.sum(seli)
                return off + cnt, eqc + jnp.sum(eqi)

            lax.fori_loop(0, S // 16, body, (jnp.int32(0), jnp.int32(0)))

            def offs(j, carry):
                v = idxraw_v[pl.ds(j * 16, 16)]
                idxoff_v[pl.ds(j * 16, 16)] = jnp.clip(v, 0, S - 1) + bconst * S
                return carry

            lax.fori_loop(0, KPAD // 16, offs, jnp.int32(0))
            pltpu.sync_copy(idxraw_v, idx_out.at[0])
            pltpu.sync_copy(nsc_v, nsc_out.at[0])
            pltpu.sync_copy(idxoff_v, shared_idx)

        @pl.when(c == 0)
        def _gather():
            plsc.subcore_barrier()
            pltpu.sync_copy(shared_idx.at[pl.ds(s * ROWS_PER_SUB, ROWS_PER_SUB)],
                            chunk_v)
            pltpu.async_copy(hs_hbm.at[chunk_v], rows_v, sem).wait()
            pltpu.sync_copy(rows_v,
                            gath_out.at[pl.ds(s * ROWS_PER_SUB, ROWS_PER_SUB)])

    return _sc_body


def _select_gather_b(bconst, scores2d, thr, quo, hs_flat):
    mesh = plsc.VectorSubcoreMesh(core_axis_name="c", subcore_axis_name="s")
    fn = pl.kernel(
        _make_sc_body(bconst),
        out_type=[
            jax.ShapeDtypeStruct((1, KPAD), jnp.int32),
            jax.ShapeDtypeStruct((1, KPAD), jnp.float32),
            jax.ShapeDtypeStruct((KPAD, D), jnp.float32),
        ],
        mesh=mesh,
        scratch_types=[
            pltpu.VMEM((S,), jnp.float32),
            pltpu.VMEM((128,), jnp.float32),
            pltpu.VMEM((128,), jnp.int32),
            pltpu.VMEM((KPAD,), jnp.int32),
            pltpu.VMEM((KPAD,), jnp.int32),
            pltpu.VMEM((KPAD,), jnp.float32),
            pltpu.VMEM_SHARED((KPAD,), jnp.int32),
            pltpu.VMEM((ROWS_PER_SUB,), jnp.int32),
            pltpu.VMEM((ROWS_PER_SUB, D), jnp.float32),
            pltpu.SemaphoreType.DMA,
        ],
        compiler_params=pltpu.CompilerParams(needs_layout_passes=False),
    )
    return fn(scores2d, thr, quo, hs_flat)


def _vffn2_body(g0_ref, g1_ref, wvt_ref, bv_ref, idx0_ref, nsc0_ref,
                idx1_ref, nsc1_ref, o_ref, oidx_ref, onsc_ref):
    i = pl.program_id(0)

    @pl.when(i == 0)
    def _b0():
        xb = g0_ref[...].astype(jnp.bfloat16)
        acc = (jnp.dot(xb, wvt_ref[...], preferred_element_type=jnp.float32)
               + bv_ref[...])
        o_ref[...] = acc[:K][None]
        oidx_ref[...] = idx0_ref[:, :, :K]
        onsc_ref[...] = nsc0_ref[:, :, :K]

    @pl.when(i == 1)
    def _b1():
        xb = g1_ref[...].astype(jnp.bfloat16)
        acc = (jnp.dot(xb, wvt_ref[...], preferred_element_type=jnp.float32)
               + bv_ref[...])
        o_ref[...] = acc[:K][None]
        oidx_ref[...] = idx1_ref[:, :, :K]
        onsc_ref[...] = nsc1_ref[:, :, :K]


def _vffn2(g0, g1, wvt, bv, idx0, nsc0, idx1, nsc1):
    return pl.pallas_call(
        _vffn2_body,
        grid=(B,),
        in_specs=[
            pl.BlockSpec((KPAD, D), lambda i: (0, 0)),
            pl.BlockSpec((KPAD, D), lambda i: (0, 0)),
            pl.BlockSpec((D, D), lambda i: (0, 0)),
            pl.BlockSpec((1, D), lambda i: (0, 0)),
            pl.BlockSpec((1, 1, KPAD), lambda i: (0, 0, 0)),
            pl.BlockSpec((1, 1, KPAD), lambda i: (0, 0, 0)),
            pl.BlockSpec((1, 1, KPAD), lambda i: (0, 0, 0)),
            pl.BlockSpec((1, 1, KPAD), lambda i: (0, 0, 0)),
        ],
        out_specs=[
            pl.BlockSpec((1, K, D), lambda i: (i, 0, 0)),
            pl.BlockSpec((1, 1, K), lambda i: (i, 0, 0)),
            pl.BlockSpec((1, 1, K), lambda i: (i, 0, 0)),
        ],
        out_shape=[
            jax.ShapeDtypeStruct((B, K, D), jnp.float32),
            jax.ShapeDtypeStruct((B, 1, K), jnp.int32),
            jax.ShapeDtypeStruct((B, 1, K), jnp.float32),
        ],
    )(g0, g1, wvt, bv, idx0, nsc0, idx1, nsc1)


@jax.jit
def kernel(input_ids, attention_mask, last_hidden, hidden_states,
           W1, b1, W2, b2, Wv, bv):
    w1tb = W1.astype(jnp.bfloat16)
    wvtb = Wv.T.astype(jnp.bfloat16)
    b1r = b1.reshape(1, D)
    w2b = W2.reshape(1, D).astype(jnp.bfloat16)
    b2r = b2.reshape(1, 1)
    hs_flat = hidden_states.reshape(B * S, D)

    per_b = []
    for b in range(B):
        sc3 = _scores_b(last_hidden[b], w1tb, b1r, w2b, b2r,
                        attention_mask[b].reshape(1, 1, S))
        sc2 = sc3.reshape(1, S)
        thr, quo = _threshold_b(sc2)
        idx_b, nsc_b, gath_b = _select_gather_b(b, sc2, thr, quo, hs_flat)
        per_b.append((sc2, idx_b, nsc_b, gath_b))

    enc_full, idx3, nsc3 = _vffn2(
        per_b[0][3], per_b[1][3], wvtb, bv.reshape(1, D),
        per_b[0][1].reshape(1, 1, KPAD), per_b[0][2].reshape(1, 1, KPAD),
        per_b[1][1].reshape(1, 1, KPAD), per_b[1][2].reshape(1, 1, KPAD))
    enc = enc_full
    indices = idx3.reshape(B, K)
    nugget_scores = nsc3.reshape(B, K)
    scores2d = jnp.concatenate([per_b[0][0], per_b[1][0]], axis=0)
    nugget_mask = jnp.ones((B, K), dtype=bool)
    return (enc, nugget_mask, nugget_scores, indices, scores2d)


# speedup vs baseline: 1.2829x; 1.2829x over previous
import functools

import jax
import jax.numpy as jnp
from jax import lax
from jax.experimental import pallas as pl
from jax.experimental.pallas import tpu as pltpu
from jax.experimental.pallas import tpu_sc as plsc

B, S, D = 2, 4096, 2048
K = 410
KPAD = 512
BS = 512
NSUB = 16
ROWS_PER_SUB = KPAD // NSUB


def _score_body(x_ref, w1t_ref, b1_ref, w2_ref, b2_ref, m_ref, o_ref):
    xb = x_ref[...].astype(jnp.bfloat16)
    h = lax.dot_general(xb, w1t_ref[...], (((1,), (1,)), ((), ())),
                        preferred_element_type=jnp.float32) + b1_ref[...]
    h = jnp.maximum(h, 0.0)
    hb = h.astype(jnp.bfloat16)
    s = lax.dot_general(w2_ref[...], hb, (((1,), (1,)), ((), ())),
                        preferred_element_type=jnp.float32)
    s = s + b2_ref[...]
    neg = jnp.finfo(jnp.float32).min
    o_ref[...] = jnp.where(m_ref[0] != 0, s, neg)[None]


def _scores(x2d, w1t, b1, w2row, b2, mask3d):
    grid = (B * S) // BS
    nsb = S // BS
    return pl.pallas_call(
        _score_body,
        grid=(grid,),
        in_specs=[
            pl.BlockSpec((BS, D), lambda i: (i, 0)),
            pl.BlockSpec((D, D), lambda i: (0, 0)),
            pl.BlockSpec((1, D), lambda i: (0, 0)),
            pl.BlockSpec((1, D), lambda i: (0, 0)),
            pl.BlockSpec((1, 1), lambda i: (0, 0)),
            pl.BlockSpec((1, 1, BS), lambda i: (i // nsb, 0, i % nsb)),
        ],
        out_specs=pl.BlockSpec((1, 1, BS), lambda i: (i // nsb, 0, i % nsb)),
        out_shape=jax.ShapeDtypeStruct((B, 1, S), jnp.float32),
        compiler_params=pltpu.CompilerParams(
            dimension_semantics=("parallel",)),
    )(x2d, w1t, b1, w2row, b2, mask3d)


def _sortable_i32(bits):
    return jnp.where(
        bits >= 0, bits, jnp.bitwise_xor(jnp.bitwise_and(bits, 0x7FFFFFFF), -1)
    )


def _thresh_body(s_ref, thr_ref, quo_ref):
    bits = lax.bitcast_convert_type(s_ref[...], jnp.int32)
    key = _sortable_i32(bits)
    imin = jnp.int32(-2147483648)

    def step(j, prefix_u):
        b = 31 - j
        cand_u = jnp.bitwise_or(prefix_u, jnp.left_shift(jnp.int32(1), b))
        cand_key = jnp.bitwise_xor(cand_u, imin)
        cnt = jnp.sum((key >= cand_key).astype(jnp.int32), axis=1, keepdims=True)
        return jnp.where(cnt >= K, cand_u, prefix_u)

    prefix_u = lax.fori_loop(0, 32, step, jnp.zeros((B, 1), jnp.int32))
    t = jnp.bitwise_xor(prefix_u, imin)
    cnt_gt = jnp.sum((key > t).astype(jnp.int32), axis=1, keepdims=True)
    quota = K - cnt_gt
    t_f = lax.bitcast_convert_type(_sortable_i32(t), jnp.float32)
    thr_ref[...] = jnp.broadcast_to(t_f, (B, 128))
    quo_ref[...] = jnp.broadcast_to(quota, (B, 128))


def _threshold(scores2d):
    return pl.pallas_call(
        _thresh_body,
        out_shape=[
            jax.ShapeDtypeStruct((B, 128), jnp.float32),
            jax.ShapeDtypeStruct((B, 128), jnp.int32),
        ],
    )(scores2d)


def _sc_body(scores_hbm, thr_hbm, quo_hbm, hs_hbm,
             idx_out, nsc_out, gath_out,
             scores_v, thr_v, quo_v, idxraw_v, idxoff_v, nsc_v,
             shared_idx, chunk_v, rows_v, sem):
    c = lax.axis_index("c")
    s = lax.axis_index("s")
    b = c

    @pl.when(s == 0)
    def _compact():
        pltpu.sync_copy(scores_hbm.at[b], scores_v)
        pltpu.sync_copy(thr_hbm.at[b], thr_v)
        pltpu.sync_copy(quo_hbm.at[b], quo_v)
        def zero(j, carry):
            idxraw_v[pl.ds(j * 16, 16)] = jnp.zeros((16,), jnp.int32)
            nsc_v[pl.ds(j * 16, 16)] = jnp.zeros((16,), jnp.float32)
            return carry

        lax.fori_loop(0, KPAD // 16, zero, jnp.int32(0))

        def body(i, carry):
            off, eqc = carry
            tvec = thr_v[pl.ds(0, 16)]
            qvec = quo_v[pl.ds(0, 16)]
            iota = lax.iota(jnp.int32, 16)
            sv = scores_v[pl.ds(i * 16, 16)]
            selgt = sv > tvec
            seleq = sv == tvec
            eqi = seleq.astype(jnp.int32)
            rank = plsc.cumsum(eqi) - eqi + eqc
            sel = jnp.logical_or(selgt, jnp.logical_and(seleq, rank < qvec))
            plsc.store_compressed(idxraw_v.at[pl.ds(off, 16)], iota + i * 16,
                                  mask=sel)
            plsc.store_compressed(nsc_v.at[pl.ds(off, 16)], sv, mask=sel)
            cnt = jnp.sum(sel.astype(jnp.int32))
            return off + cnt, eqc + jnp.sum(eqi)

        lax.fori_loop(0, S // 16, body, (jnp.int32(0), jnp.int32(0)))

        def offs(j, carry):
            v = idxraw_v[pl.ds(j * 16, 16)]
            idxoff_v[pl.ds(j * 16, 16)] = jnp.clip(v, 0, S - 1) + b * S
            return carry

        lax.fori_loop(0, KPAD // 16, offs, jnp.int32(0))
        pltpu.sync_copy(idxraw_v, idx_out.at[b])
        pltpu.sync_copy(nsc_v, nsc_out.at[b])
        pltpu.sync_copy(idxoff_v, shared_idx)

    plsc.subcore_barrier()
    pltpu.sync_copy(shared_idx.at[pl.ds(s * ROWS_PER_SUB, ROWS_PER_SUB)], chunk_v)
    pltpu.async_copy(hs_hbm.at[chunk_v], rows_v, sem).wait()
    pltpu.sync_copy(rows_v,
                    gath_out.at[pl.ds(b * KPAD + s * ROWS_PER_SUB, ROWS_PER_SUB)])


def _select_gather(scores2d, thr, quo, hs_flat):
    mesh = plsc.VectorSubcoreMesh(core_axis_name="c", subcore_axis_name="s")
    fn = pl.kernel(
        _sc_body,
        out_type=[
            jax.ShapeDtypeStruct((B, KPAD), jnp.int32),
            jax.ShapeDtypeStruct((B, KPAD), jnp.float32),
            jax.ShapeDtypeStruct((B * KPAD, D), jnp.float32),
        ],
        mesh=mesh,
        scratch_types=[
            pltpu.VMEM((S,), jnp.float32),
            pltpu.VMEM((128,), jnp.float32),
            pltpu.VMEM((128,), jnp.int32),
            pltpu.VMEM((KPAD,), jnp.int32),
            pltpu.VMEM((KPAD,), jnp.int32),
            pltpu.VMEM((KPAD,), jnp.float32),
            pltpu.VMEM_SHARED((KPAD,), jnp.int32),
            pltpu.VMEM((ROWS_PER_SUB,), jnp.int32),
            pltpu.VMEM((ROWS_PER_SUB, D), jnp.float32),
            pltpu.SemaphoreType.DMA,
        ],
        compiler_params=pltpu.CompilerParams(needs_layout_passes=False),
    )
    return fn(scores2d, thr, quo, hs_flat)


def _vffn_body(x_ref, wvt_ref, bv_ref, idx_ref, nsc_ref,
               o_ref, oidx_ref, onsc_ref):
    xb = x_ref[0].astype(jnp.bfloat16)
    acc = (jnp.dot(xb, wvt_ref[...], preferred_element_type=jnp.float32)
           + bv_ref[...])
    o_ref[...] = acc[:K][None]
    oidx_ref[...] = idx_ref[:, :, :K]
    onsc_ref[...] = nsc_ref[:, :, :K]


def _vffn(gathered3d, wvt, bv, idx3, nsc3):
    return pl.pallas_call(
        _vffn_body,
        grid=(B,),
        in_specs=[
            pl.BlockSpec((1, KPAD, D), lambda i: (i, 0, 0)),
            pl.BlockSpec((D, D), lambda i: (0, 0)),
            pl.BlockSpec((1, D), lambda i: (0, 0)),
            pl.BlockSpec((1, 1, KPAD), lambda i: (i, 0, 0)),
            pl.BlockSpec((1, 1, KPAD), lambda i: (i, 0, 0)),
        ],
        out_specs=[
            pl.BlockSpec((1, K, D), lambda i: (i, 0, 0)),
            pl.BlockSpec((1, 1, K), lambda i: (i, 0, 0)),
            pl.BlockSpec((1, 1, K), lambda i: (i, 0, 0)),
        ],
        out_shape=[
            jax.ShapeDtypeStruct((B, K, D), jnp.float32),
            jax.ShapeDtypeStruct((B, 1, K), jnp.int32),
            jax.ShapeDtypeStruct((B, 1, K), jnp.float32),
        ],
        compiler_params=pltpu.CompilerParams(
            dimension_semantics=("parallel",)),
    )(gathered3d, wvt, bv, idx3, nsc3)


@jax.jit
def kernel(input_ids, attention_mask, last_hidden, hidden_states,
           W1, b1, W2, b2, Wv, bv):
    x2d = last_hidden.reshape(B * S, D)
    mask3d = attention_mask.reshape(B, 1, S)
    scores3d = _scores(x2d, W1.astype(jnp.bfloat16), b1.reshape(1, D),
                       W2.reshape(1, D).astype(jnp.bfloat16),
                       b2.reshape(1, 1), mask3d)
    scores2d = scores3d.reshape(B, S)
    thr, quo = _threshold(scores2d)
    hs_flat = hidden_states.reshape(B * S, D)
    idx_pad, nsc_pad, gathered = _select_gather(scores2d, thr, quo, hs_flat)
    enc, idx3, nsc3 = _vffn(gathered.reshape(B, KPAD, D),
                            Wv.T.astype(jnp.bfloat16), bv.reshape(1, D),
                            idx_pad.reshape(B, 1, KPAD),
                            nsc_pad.reshape(B, 1, KPAD))
    indices = idx3.reshape(B, K)
    nugget_scores = nsc3.reshape(B, K)
    nugget_mask = jnp.ones((B, K), dtype=bool)
    return (enc, nugget_mask, nugget_scores, indices, scores2d)


# speedup vs baseline: 1.2878x; 1.0038x over previous
import functools

import jax
import jax.numpy as jnp
from jax import lax
from jax.experimental import pallas as pl
from jax.experimental.pallas import tpu as pltpu
from jax.experimental.pallas import tpu_sc as plsc

B, S, D = 2, 4096, 2048
K = 410
KPAD = 512
BS = 1024
NSUB = 16
ROWS_PER_SUB = KPAD // NSUB


def _score_body(x_ref, w1t_ref, b1_ref, w2_ref, b2_ref, m_ref, o_ref):
    xb = x_ref[...].astype(jnp.bfloat16)
    h = lax.dot_general(xb, w1t_ref[...], (((1,), (1,)), ((), ())),
                        preferred_element_type=jnp.float32) + b1_ref[...]
    h = jnp.maximum(h, 0.0)
    hb = h.astype(jnp.bfloat16)
    s = lax.dot_general(w2_ref[...], hb, (((1,), (1,)), ((), ())),
                        preferred_element_type=jnp.float32)
    s = s + b2_ref[...]
    neg = jnp.finfo(jnp.float32).min
    o_ref[...] = jnp.where(m_ref[0] != 0, s, neg)[None]


def _scores(x2d, w1t, b1, w2row, b2, mask3d):
    grid = (B * S) // BS
    nsb = S // BS
    return pl.pallas_call(
        _score_body,
        grid=(grid,),
        in_specs=[
            pl.BlockSpec((BS, D), lambda i: (i, 0)),
            pl.BlockSpec((D, D), lambda i: (0, 0)),
            pl.BlockSpec((1, D), lambda i: (0, 0)),
            pl.BlockSpec((1, D), lambda i: (0, 0)),
            pl.BlockSpec((1, 1), lambda i: (0, 0)),
            pl.BlockSpec((1, 1, BS), lambda i: (i // nsb, 0, i % nsb)),
        ],
        out_specs=pl.BlockSpec((1, 1, BS), lambda i: (i // nsb, 0, i % nsb)),
        out_shape=jax.ShapeDtypeStruct((B, 1, S), jnp.float32),
        compiler_params=pltpu.CompilerParams(
            dimension_semantics=("parallel",)),
    )(x2d, w1t, b1, w2row, b2, mask3d)


def _sortable_i32(bits):
    return jnp.where(
        bits >= 0, bits, jnp.bitwise_xor(jnp.bitwise_and(bits, 0x7FFFFFFF), -1)
    )


def _thresh_body(s_ref, thr_ref, quo_ref):
    bits = lax.bitcast_convert_type(s_ref[...], jnp.int32)
    key = _sortable_i32(bits)
    imin = jnp.int32(-2147483648)

    def step(j, prefix_u):
        b = 31 - j
        cand_u = jnp.bitwise_or(prefix_u, jnp.left_shift(jnp.int32(1), b))
        cand_key = jnp.bitwise_xor(cand_u, imin)
        cnt = jnp.sum((key >= cand_key).astype(jnp.int32), axis=1, keepdims=True)
        return jnp.where(cnt >= K, cand_u, prefix_u)

    prefix_u = lax.fori_loop(0, 32, step, jnp.zeros((B, 1), jnp.int32))
    t = jnp.bitwise_xor(prefix_u, imin)
    cnt_gt = jnp.sum((key > t).astype(jnp.int32), axis=1, keepdims=True)
    quota = K - cnt_gt
    t_f = lax.bitcast_convert_type(_sortable_i32(t), jnp.float32)
    thr_ref[...] = jnp.broadcast_to(t_f, (B, 128))
    quo_ref[...] = jnp.broadcast_to(quota, (B, 128))


def _threshold(scores2d):
    return pl.pallas_call(
        _thresh_body,
        out_shape=[
            jax.ShapeDtypeStruct((B, 128), jnp.float32),
            jax.ShapeDtypeStruct((B, 128), jnp.int32),
        ],
    )(scores2d)


def _sc_body(scores_hbm, thr_hbm, quo_hbm, hs_hbm,
             idx_out, nsc_out, gath_out,
             scores_v, thr_v, quo_v, idxraw_v, idxoff_v, nsc_v,
             shared_idx, chunk_v, rows_v, sem):
    c = lax.axis_index("c")
    s = lax.axis_index("s")
    b = c

    @pl.when(s == 0)
    def _compact():
        pltpu.sync_copy(scores_hbm.at[b], scores_v)
        pltpu.sync_copy(thr_hbm.at[b], thr_v)
        pltpu.sync_copy(quo_hbm.at[b], quo_v)
        def zero(j, carry):
            idxraw_v[pl.ds(j * 16, 16)] = jnp.zeros((16,), jnp.int32)
            nsc_v[pl.ds(j * 16, 16)] = jnp.zeros((16,), jnp.float32)
            return carry

        lax.fori_loop(0, KPAD // 16, zero, jnp.int32(0))

        def body(i, carry):
            off, eqc = carry
            tvec = thr_v[pl.ds(0, 16)]
            qvec = quo_v[pl.ds(0, 16)]
            iota = lax.iota(jnp.int32, 16)
            sv = scores_v[pl.ds(i * 16, 16)]
            selgt = sv > tvec
            seleq = sv == tvec
            eqi = seleq.astype(jnp.int32)
            rank = plsc.cumsum(eqi) - eqi + eqc
            sel = jnp.logical_or(selgt, jnp.logical_and(seleq, rank < qvec))
            plsc.store_compressed(idxraw_v.at[pl.ds(off, 16)], iota + i * 16,
                                  mask=sel)
            plsc.store_compressed(nsc_v.at[pl.ds(off, 16)], sv, mask=sel)
            cnt = jnp.sum(sel.astype(jnp.int32))
            return off + cnt, eqc + jnp.sum(eqi)

        lax.fori_loop(0, S // 16, body, (jnp.int32(0), jnp.int32(0)))

        def offs(j, carry):
            v = idxraw_v[pl.ds(j * 16, 16)]
            idxoff_v[pl.ds(j * 16, 16)] = jnp.clip(v, 0, S - 1) + b * S
            return carry

        lax.fori_loop(0, KPAD // 16, offs, jnp.int32(0))
        pltpu.sync_copy(idxraw_v, idx_out.at[b])
        pltpu.sync_copy(nsc_v, nsc_out.at[b])
        pltpu.sync_copy(idxoff_v, shared_idx)

    plsc.subcore_barrier()
    pltpu.sync_copy(shared_idx.at[pl.ds(s * ROWS_PER_SUB, ROWS_PER_SUB)], chunk_v)
    pltpu.async_copy(hs_hbm.at[chunk_v], rows_v, sem).wait()
    pltpu.sync_copy(rows_v,
                    gath_out.at[pl.ds(b * KPAD + s * ROWS_PER_SUB, ROWS_PER_SUB)])


def _select_gather(scores2d, thr, quo, hs_flat):
    mesh = plsc.VectorSubcoreMesh(core_axis_name="c", subcore_axis_name="s")
    fn = pl.kernel(
        _sc_body,
        out_type=[
            jax.ShapeDtypeStruct((B, KPAD), jnp.int32),
            jax.ShapeDtypeStruct((B, KPAD), jnp.float32),
            jax.ShapeDtypeStruct((B * KPAD, D), jnp.float32),
        ],
        mesh=mesh,
        scratch_types=[
            pltpu.VMEM((S,), jnp.float32),
            pltpu.VMEM((128,), jnp.float32),
            pltpu.VMEM((128,), jnp.int32),
            pltpu.VMEM((KPAD,), jnp.int32),
            pltpu.VMEM((KPAD,), jnp.int32),
            pltpu.VMEM((KPAD,), jnp.float32),
            pltpu.VMEM_SHARED((KPAD,), jnp.int32),
            pltpu.VMEM((ROWS_PER_SUB,), jnp.int32),
            pltpu.VMEM((ROWS_PER_SUB, D), jnp.float32),
            pltpu.SemaphoreType.DMA,
        ],
        compiler_params=pltpu.CompilerParams(needs_layout_passes=False),
    )
    return fn(scores2d, thr, quo, hs_flat)


def _vffn_body(x_ref, wvt_ref, bv_ref, idx_ref, nsc_ref,
               o_ref, oidx_ref, onsc_ref):
    xb = x_ref[0].astype(jnp.bfloat16)
    acc = (jnp.dot(xb, wvt_ref[...], preferred_element_type=jnp.float32)
           + bv_ref[...])
    o_ref[...] = acc[:K][None]
    oidx_ref[...] = idx_ref[:, :, :K]
    onsc_ref[...] = nsc_ref[:, :, :K]


def _vffn(gathered3d, wvt, bv, idx3, nsc3):
    return pl.pallas_call(
        _vffn_body,
        grid=(B,),
        in_specs=[
            pl.BlockSpec((1, KPAD, D), lambda i: (i, 0, 0)),
            pl.BlockSpec((D, D), lambda i: (0, 0)),
            pl.BlockSpec((1, D), lambda i: (0, 0)),
            pl.BlockSpec((1, 1, KPAD), lambda i: (i, 0, 0)),
            pl.BlockSpec((1, 1, KPAD), lambda i: (i, 0, 0)),
        ],
        out_specs=[
            pl.BlockSpec((1, K, D), lambda i: (i, 0, 0)),
            pl.BlockSpec((1, 1, K), lambda i: (i, 0, 0)),
            pl.BlockSpec((1, 1, K), lambda i: (i, 0, 0)),
        ],
        out_shape=[
            jax.ShapeDtypeStruct((B, K, D), jnp.float32),
            jax.ShapeDtypeStruct((B, 1, K), jnp.int32),
            jax.ShapeDtypeStruct((B, 1, K), jnp.float32),
        ],
        compiler_params=pltpu.CompilerParams(
            dimension_semantics=("parallel",)),
    )(gathered3d, wvt, bv, idx3, nsc3)


@jax.jit
def kernel(input_ids, attention_mask, last_hidden, hidden_states,
           W1, b1, W2, b2, Wv, bv):
    x2d = last_hidden.reshape(B * S, D)
    mask3d = attention_mask.reshape(B, 1, S)
    scores3d = _scores(x2d, W1.astype(jnp.bfloat16), b1.reshape(1, D),
                       W2.reshape(1, D).astype(jnp.bfloat16),
                       b2.reshape(1, 1), mask3d)
    scores2d = scores3d.reshape(B, S)
    thr, quo = _threshold(scores2d)
    hs_flat = hidden_states.reshape(B * S, D)
    idx_pad, nsc_pad, gathered = _select_gather(scores2d, thr, quo, hs_flat)
    enc, idx3, nsc3 = _vffn(gathered.reshape(B, KPAD, D),
                            Wv.T.astype(jnp.bfloat16), bv.reshape(1, D),
                            idx_pad.reshape(B, 1, KPAD),
                            nsc_pad.reshape(B, 1, KPAD))
    indices = idx3.reshape(B, K)
    nugget_scores = nsc3.reshape(B, K)
    nugget_mask = jnp.ones((B, K), dtype=bool)
    return (enc, nugget_mask, nugget_scores, indices, scores2d)


# speedup vs baseline: 1.2899x; 1.0016x over previous
import functools

import jax
import jax.numpy as jnp
from jax import lax
from jax.experimental import pallas as pl
from jax.experimental.pallas import tpu as pltpu
from jax.experimental.pallas import tpu_sc as plsc

B, S, D = 2, 4096, 2048
K = 410
KPAD = 512
BS = 1024
NSUB = 16
ROWS_PER_SUB = KPAD // NSUB


def _score_body(x_ref, w1t_ref, b1_ref, w2_ref, b2_ref, m_ref, o_ref):
    xb = x_ref[...].astype(jnp.bfloat16)
    h = lax.dot_general(xb, w1t_ref[...], (((1,), (1,)), ((), ())),
                        preferred_element_type=jnp.float32) + b1_ref[...]
    h = jnp.maximum(h, 0.0)
    hb = h.astype(jnp.bfloat16)
    s = lax.dot_general(w2_ref[...], hb, (((1,), (1,)), ((), ())),
                        preferred_element_type=jnp.float32)
    s = s + b2_ref[...]
    neg = jnp.finfo(jnp.float32).min
    o_ref[...] = jnp.where(m_ref[0] != 0, s, neg)[None]


def _scores(x2d, w1t, b1, w2row, b2, mask3d):
    grid = (B * S) // BS
    nsb = S // BS
    return pl.pallas_call(
        _score_body,
        grid=(grid,),
        in_specs=[
            pl.BlockSpec((BS, D), lambda i: (i, 0)),
            pl.BlockSpec((D, D), lambda i: (0, 0)),
            pl.BlockSpec((1, D), lambda i: (0, 0)),
            pl.BlockSpec((1, D), lambda i: (0, 0)),
            pl.BlockSpec((1, 1), lambda i: (0, 0)),
            pl.BlockSpec((1, 1, BS), lambda i: (i // nsb, 0, i % nsb)),
        ],
        out_specs=pl.BlockSpec((1, 1, BS), lambda i: (i // nsb, 0, i % nsb)),
        out_shape=jax.ShapeDtypeStruct((B, 1, S), jnp.float32),
        compiler_params=pltpu.CompilerParams(
            dimension_semantics=("parallel",)),
    )(x2d, w1t, b1, w2row, b2, mask3d)


def _sortable_i32(bits):
    return jnp.where(
        bits >= 0, bits, jnp.bitwise_xor(jnp.bitwise_and(bits, 0x7FFFFFFF), -1)
    )


def _thresh_body(s_ref, thr_ref, quo_ref):
    bits = lax.bitcast_convert_type(s_ref[...], jnp.int32)
    key = _sortable_i32(bits)
    imin = jnp.int32(-2147483648)

    def step(j, prefix_u):
        b = 31 - j
        cand_u = jnp.bitwise_or(prefix_u, jnp.left_shift(jnp.int32(1), b))
        cand_key = jnp.bitwise_xor(cand_u, imin)
        cnt = jnp.sum((key >= cand_key).astype(jnp.int32), axis=1, keepdims=True)
        return jnp.where(cnt >= K, cand_u, prefix_u)

    prefix_u = lax.fori_loop(0, 32, step, jnp.zeros((B, 1), jnp.int32))
    t = jnp.bitwise_xor(prefix_u, imin)
    cnt_gt = jnp.sum((key > t).astype(jnp.int32), axis=1, keepdims=True)
    quota = K - cnt_gt
    t_f = lax.bitcast_convert_type(_sortable_i32(t), jnp.float32)
    thr_ref[...] = jnp.broadcast_to(t_f, (B, 128))
    quo_ref[...] = jnp.broadcast_to(quota, (B, 128))


def _threshold(scores2d):
    return pl.pallas_call(
        _thresh_body,
        out_shape=[
            jax.ShapeDtypeStruct((B, 128), jnp.float32),
            jax.ShapeDtypeStruct((B, 128), jnp.int32),
        ],
    )(scores2d)


def _sc_body(scores_hbm, thr_hbm, quo_hbm, hs_hbm,
             idx_out, nsc_out, gath_out,
             scores_v, thr_v, quo_v, idxraw_v, idxoff_v, nsc_v,
             shared_idx, chunk_v, rows_v, sem):
    c = lax.axis_index("c")
    s = lax.axis_index("s")
    b = c

    @pl.when(s == 0)
    def _compact():
        pltpu.sync_copy(scores_hbm.at[b], scores_v)
        pltpu.sync_copy(thr_hbm.at[b], thr_v)
        pltpu.sync_copy(quo_hbm.at[b], quo_v)
        def zero(j, carry):
            idxraw_v[pl.ds(j * 16, 16)] = jnp.zeros((16,), jnp.int32)
            nsc_v[pl.ds(j * 16, 16)] = jnp.zeros((16,), jnp.float32)
            return carry

        lax.fori_loop(0, KPAD // 16, zero, jnp.int32(0))

        tvec = thr_v[pl.ds(0, 16)]
        qvec = quo_v[pl.ds(0, 16)]
        iota = lax.iota(jnp.int32, 16)

        def chunk(base, off, eqc):
            sv = scores_v[pl.ds(base, 16)]
            selgt = sv > tvec
            seleq = sv == tvec
            eqi = seleq.astype(jnp.int32)
            rank = plsc.cumsum(eqi) - eqi + eqc
            sel = jnp.logical_or(selgt, jnp.logical_and(seleq, rank < qvec))
            plsc.store_compressed(idxraw_v.at[pl.ds(off, 16)], iota + base,
                                  mask=sel)
            plsc.store_compressed(nsc_v.at[pl.ds(off, 16)], sv, mask=sel)
            cnt = jnp.sum(sel.astype(jnp.int32))
            return off + cnt, eqc + jnp.sum(eqi)

        def body(i, carry):
            off, eqc = carry
            off, eqc = chunk(i * 32, off, eqc)
            off, eqc = chunk(i * 32 + 16, off, eqc)
            return off, eqc

        lax.fori_loop(0, S // 32, body, (jnp.int32(0), jnp.int32(0)))

        def offs(j, carry):
            v = idxraw_v[pl.ds(j * 16, 16)]
            idxoff_v[pl.ds(j * 16, 16)] = jnp.clip(v, 0, S - 1) + b * S
            return carry

        lax.fori_loop(0, KPAD // 16, offs, jnp.int32(0))
        pltpu.sync_copy(idxraw_v, idx_out.at[b])
        pltpu.sync_copy(nsc_v, nsc_out.at[b])
        pltpu.sync_copy(idxoff_v, shared_idx)

    plsc.subcore_barrier()
    pltpu.sync_copy(shared_idx.at[pl.ds(s * ROWS_PER_SUB, ROWS_PER_SUB)], chunk_v)
    pltpu.async_copy(hs_hbm.at[chunk_v], rows_v, sem).wait()
    pltpu.sync_copy(rows_v,
                    gath_out.at[pl.ds(b * KPAD + s * ROWS_PER_SUB, ROWS_PER_SUB)])


def _select_gather(scores2d, thr, quo, hs_flat):
    mesh = plsc.VectorSubcoreMesh(core_axis_name="c", subcore_axis_name="s")
    fn = pl.kernel(
        _sc_body,
        out_type=[
            jax.ShapeDtypeStruct((B, KPAD), jnp.int32),
            jax.ShapeDtypeStruct((B, KPAD), jnp.float32),
            jax.ShapeDtypeStruct((B * KPAD, D), jnp.float32),
        ],
        mesh=mesh,
        scratch_types=[
            pltpu.VMEM((S,), jnp.float32),
            pltpu.VMEM((128,), jnp.float32),
            pltpu.VMEM((128,), jnp.int32),
            pltpu.VMEM((KPAD,), jnp.int32),
            pltpu.VMEM((KPAD,), jnp.int32),
            pltpu.VMEM((KPAD,), jnp.float32),
            pltpu.VMEM_SHARED((KPAD,), jnp.int32),
            pltpu.VMEM((ROWS_PER_SUB,), jnp.int32),
            pltpu.VMEM((ROWS_PER_SUB, D), jnp.float32),
            pltpu.SemaphoreType.DMA,
        ],
        compiler_params=pltpu.CompilerParams(needs_layout_passes=False),
    )
    return fn(scores2d, thr, quo, hs_flat)


def _vffn_body(x_ref, wvt_ref, bv_ref, idx_ref, nsc_ref,
               o_ref, oidx_ref, onsc_ref):
    xb = x_ref[0].astype(jnp.bfloat16)
    acc = (jnp.dot(xb, wvt_ref[...], preferred_element_type=jnp.float32)
           + bv_ref[...])
    o_ref[...] = acc[:K][None]
    oidx_ref[...] = idx_ref[:, :, :K]
    onsc_ref[...] = nsc_ref[:, :, :K]


def _vffn(gathered3d, wvt, bv, idx3, nsc3):
    return pl.pallas_call(
        _vffn_body,
        grid=(B,),
        in_specs=[
            pl.BlockSpec((1, KPAD, D), lambda i: (i, 0, 0)),
            pl.BlockSpec((D, D), lambda i: (0, 0)),
            pl.BlockSpec((1, D), lambda i: (0, 0)),
            pl.BlockSpec((1, 1, KPAD), lambda i: (i, 0, 0)),
            pl.BlockSpec((1, 1, KPAD), lambda i: (i, 0, 0)),
        ],
        out_specs=[
            pl.BlockSpec((1, K, D), lambda i: (i, 0, 0)),
            pl.BlockSpec((1, 1, K), lambda i: (i, 0, 0)),
            pl.BlockSpec((1, 1, K), lambda i: (i, 0, 0)),
        ],
        out_shape=[
            jax.ShapeDtypeStruct((B, K, D), jnp.float32),
            jax.ShapeDtypeStruct((B, 1, K), jnp.int32),
            jax.ShapeDtypeStruct((B, 1, K), jnp.float32),
        ],
        compiler_params=pltpu.CompilerParams(
            dimension_semantics=("parallel",)),
    )(gathered3d, wvt, bv, idx3, nsc3)


@jax.jit
def kernel(input_ids, attention_mask, last_hidden, hidden_states,
           W1, b1, W2, b2, Wv, bv):
    x2d = last_hidden.reshape(B * S, D)
    mask3d = attention_mask.reshape(B, 1, S)
    scores3d = _scores(x2d, W1.astype(jnp.bfloat16), b1.reshape(1, D),
                       W2.reshape(1, D).astype(jnp.bfloat16),
                       b2.reshape(1, 1), mask3d)
    scores2d = scores3d.reshape(B, S)
    thr, quo = _threshold(scores2d)
    hs_flat = hidden_states.reshape(B * S, D)
    idx_pad, nsc_pad, gathered = _select_gather(scores2d, thr, quo, hs_flat)
    enc, idx3, nsc3 = _vffn(gathered.reshape(B, KPAD, D),
                            Wv.T.astype(jnp.bfloat16), bv.reshape(1, D),
                            idx_pad.reshape(B, 1, KPAD),
                            nsc_pad.reshape(B, 1, KPAD))
    indices = idx3.reshape(B, K)
    nugget_scores = nsc3.reshape(B, K)
    nugget_mask = jnp.ones((B, K), dtype=bool)
    return (enc, nugget_mask, nugget_scores, indices, scores2d)
